# baseline (speedup 1.0000x reference)
import functools

import jax
import jax.numpy as jnp
from jax.experimental import pallas as pl
from jax.experimental.pallas import tpu as pltpu

NUM_DIVISION = 10
NUM_NODES = 100000
D_FEAT = 128
TAU = 1.0
CHUNK = NUM_NODES // NUM_DIVISION


def _copy_scale_kernel(idx_ref, scale_ref, x_hbm, out_hbm, outidx_ref,
                       slab_vmem, copy_sems, slab_sem):
    idx = idx_ref[0]
    outidx_ref[...] = idx * CHUNK + jax.lax.broadcasted_iota(
        jnp.int32, (1, CHUNK), 1
    )

    pltpu.make_async_copy(
        x_hbm.at[pl.ds(idx * CHUNK, CHUNK), :], slab_vmem, slab_sem
    ).start()

    for c in range(NUM_DIVISION):
        @pl.when(idx != c)
        def _():
            pltpu.make_async_copy(
                x_hbm.at[pl.ds(c * CHUNK, CHUNK), :],
                out_hbm.at[pl.ds(c * CHUNK, CHUNK), :],
                copy_sems.at[c],
            ).start()

    pltpu.make_async_copy(
        x_hbm.at[pl.ds(idx * CHUNK, CHUNK), :], slab_vmem, slab_sem
    ).wait()
    slab_vmem[...] = slab_vmem[...] * scale_ref[0]
    slab_out = pltpu.make_async_copy(
        slab_vmem, out_hbm.at[pl.ds(idx * CHUNK, CHUNK), :], slab_sem
    )
    slab_out.start()
    slab_out.wait()

    for c in range(NUM_DIVISION):
        @pl.when(idx != c)
        def _():
            pltpu.make_async_copy(
                x_hbm.at[pl.ds(c * CHUNK, CHUNK), :],
                out_hbm.at[pl.ds(c * CHUNK, CHUNK), :],
                copy_sems.at[c],
            ).wait()


@functools.partial(jax.jit, static_argnames=("interpret",))
def kernel(node_features, uniform_noise, sample_weights, interpret=False):
    g = -jnp.log(-jnp.log(uniform_noise))
    y_soft = jax.nn.softmax((sample_weights + g) / TAU, axis=-1)
    idx = jnp.argmax(y_soft, axis=-1).astype(jnp.int32)
    y = (1.0 + y_soft[idx]) - y_soft[idx]
    idx_arr = idx[None]
    scale_arr = y[None].astype(jnp.float32)

    updated, outidx = pl.pallas_call(
        _copy_scale_kernel,
        in_specs=[
            pl.BlockSpec(memory_space=pltpu.SMEM),
            pl.BlockSpec(memory_space=pltpu.SMEM),
            pl.BlockSpec(memory_space=pltpu.MemorySpace.HBM),
        ],
        out_specs=[
            pl.BlockSpec(memory_space=pltpu.MemorySpace.HBM),
            pl.BlockSpec(memory_space=pltpu.VMEM),
        ],
        out_shape=[
            jax.ShapeDtypeStruct((NUM_NODES, D_FEAT), jnp.float32),
            jax.ShapeDtypeStruct((1, CHUNK), jnp.int32),
        ],
        scratch_shapes=[
            pltpu.VMEM((CHUNK, D_FEAT), jnp.float32),
            pltpu.SemaphoreType.DMA((NUM_DIVISION,)),
            pltpu.SemaphoreType.DMA,
        ],
        interpret=interpret,
    )(idx_arr, scale_arr, node_features)

    return updated, outidx.reshape(CHUNK)


# speedup vs baseline: 31.1414x; 31.1414x over previous
import functools

import jax
import jax.numpy as jnp
from jax.experimental import pallas as pl
from jax.experimental.pallas import tpu as pltpu

NUM_DIVISION = 10
NUM_NODES = 100000
D_FEAT = 128
TAU = 1.0
CHUNK = NUM_NODES // NUM_DIVISION

BLOCK_ROWS = 10000
NUM_BLOCKS = NUM_NODES // BLOCK_ROWS
BLOCKS_PER_CHUNK = CHUNK // BLOCK_ROWS
IDX_PER_STEP = CHUNK // NUM_BLOCKS


def _copy_scale_kernel(idx_ref, scale_ref, x_ref, out_ref, outidx_ref):
    i = pl.program_id(0)
    outidx_ref[...] = (
        idx_ref[0] * CHUNK
        + i * IDX_PER_STEP
        + jax.lax.broadcasted_iota(jnp.int32, (1, 1, IDX_PER_STEP), 2)
    )
    in_slab = (i // BLOCKS_PER_CHUNK) == idx_ref[0]
    w = jnp.where(in_slab, scale_ref[0], jnp.float32(1.0))
    out_ref[...] = x_ref[...] * w


@functools.partial(jax.jit, static_argnames=("interpret",))
def kernel(node_features, uniform_noise, sample_weights, interpret=False):
    g = -jnp.log(-jnp.log(uniform_noise))
    y_soft = jax.nn.softmax((sample_weights + g) / TAU, axis=-1)
    idx = jnp.argmax(y_soft, axis=-1).astype(jnp.int32)
    y = (1.0 + y_soft[idx]) - y_soft[idx]
    idx_arr = idx[None]
    scale_arr = y[None].astype(jnp.float32)

    updated, outidx = pl.pallas_call(
        _copy_scale_kernel,
        grid=(NUM_BLOCKS,),
        in_specs=[
            pl.BlockSpec(memory_space=pltpu.SMEM),
            pl.BlockSpec(memory_space=pltpu.SMEM),
            pl.BlockSpec((BLOCK_ROWS, D_FEAT), lambda i: (i, 0)),
        ],
        out_specs=[
            pl.BlockSpec((BLOCK_ROWS, D_FEAT), lambda i: (i, 0)),
            pl.BlockSpec((1, 1, IDX_PER_STEP), lambda i: (i, 0, 0)),
        ],
        out_shape=[
            jax.ShapeDtypeStruct((NUM_NODES, D_FEAT), jnp.float32),
            jax.ShapeDtypeStruct((NUM_BLOCKS, 1, IDX_PER_STEP), jnp.int32),
        ],
        compiler_params=pltpu.CompilerParams(
            dimension_semantics=("parallel",),
        ),
        interpret=interpret,
    )(idx_arr, scale_arr, node_features)

    return updated, outidx.reshape(CHUNK)
